# baseline (speedup 1.0000x reference)
import functools

import jax
import jax.numpy as jnp
from jax import lax
from jax.experimental import pallas as pl
from jax.experimental.pallas import tpu as pltpu
from jax.experimental.pallas import tpu_sc as plsc

DIM = 4096
BATCH = 4096
L = 16

NC = 2
NS = 16
NW = NC * NS
ROWS_PER_W = BATCH // NW
RB = 8
NB = ROWS_PER_W // RB
NP = NB // 2
HD = DIM // 2

_mesh = plsc.VectorSubcoreMesh(core_axis_name="c", subcore_axis_name="s")


@functools.partial(
    pl.kernel,
    out_type=jax.ShapeDtypeStruct((BATCH, DIM), jnp.float32),
    mesh=_mesh,
    scratch_types=[
        pltpu.VMEM((DIM,), jnp.int32),
        pltpu.VMEM((RB, DIM), jnp.float32),
        pltpu.VMEM((RB, DIM), jnp.float32),
        pltpu.VMEM((RB, HD), jnp.float32),
        pltpu.VMEM((RB, HD), jnp.float32),
        pltpu.SemaphoreType.DMA,
        pltpu.SemaphoreType.DMA,
        pltpu.SemaphoreType.DMA,
        pltpu.SemaphoreType.DMA,
    ],
    compiler_params=pltpu.CompilerParams(
        use_tc_tiling_on_sc=True, needs_layout_passes=False
    ),
)
def _permute(x_hbm, perm_hbm, out_hbm, perm_v, in0, in1, outa, outb,
             si0, si1, soa, sob):
    wid = lax.axis_index("s") * NC + lax.axis_index("c")
    row0 = wid * ROWS_PER_W

    ins = (in0, in1)
    outs = (outa, outb)
    sin = (si0, si1)
    sout = (soa, sob)

    def start_in(b, slot):
        pltpu.make_async_copy(
            x_hbm.at[pl.ds(row0 + b * RB, RB)], ins[slot], sin[slot]).start()

    def wait_in(slot):
        pltpu.make_async_copy(
            x_hbm.at[pl.ds(0, RB)], ins[slot], sin[slot]).wait()

    def start_out(b, h):
        pltpu.make_async_copy(
            outs[h],
            out_hbm.at[pl.ds(row0 + b * RB, RB), pl.ds(h * HD, HD)],
            sout[h]).start()

    def wait_out(h):
        pltpu.make_async_copy(
            outs[h],
            out_hbm.at[pl.ds(0, RB), pl.ds(h * HD, HD)],
            sout[h]).wait()

    def gather_half(src, h):
        @plsc.parallel_loop(h * HD, (h + 1) * HD, step=L, unroll=1)
        def _jloop(j):
            pv = perm_v[pl.ds(j, L)]
            for r in range(RB):
                rsel = jnp.full((L,), r, jnp.int32)
                outs[h][r, pl.ds(j - h * HD, L)] = plsc.load_gather(
                    src, [rsel, pv])

    start_in(0, 0)
    start_in(1, 1)
    pltpu.sync_copy(perm_hbm, perm_v)

    def pair_body(k, carry):
        b0 = 2 * k
        wait_in(0)
        for h in range(2):
            @pl.when(k > 0)
            def _():
                wait_out(h)
            gather_half(ins[0], h)
            start_out(b0, h)

        @pl.when(k < NP - 1)
        def _():
            start_in(b0 + 2, 0)

        wait_in(1)
        for h in range(2):
            wait_out(h)
            gather_half(ins[1], h)
            start_out(b0 + 1, h)

        @pl.when(k < NP - 1)
        def _():
            start_in(b0 + 3, 1)
        return carry

    lax.fori_loop(0, NP, pair_body, 0)
    wait_out(0)
    wait_out(1)


def kernel(x, perm):
    return _permute(x, perm)
